# Initial kernel scaffold; baseline (speedup 1.0000x reference)
import jax
import jax.numpy as jnp
from jax.experimental import pallas as pl


def kernel(x, edge_index, wq_mu, wq_lv, bq_mu, bq_lv, wk_mu, wk_lv, bk_mu, bk_lv, wv_mu, wv_lv, bv_mu, bv_lv, att):
    raise NotImplementedError("write your pallas kernel here")


# speedup vs baseline: 50.7748x; 50.7748x over previous
import functools

import jax
import jax.numpy as jnp
from jax import lax
from jax.experimental import pallas as pl
from jax.experimental.pallas import tpu as pltpu
from jax.experimental.pallas import tpu_sc as plsc

_H = 4
_C = 32


def _proj_body(x_ref, wq_ref, bq_ref, wk_ref, bk_ref, wv_ref, bv_ref,
               aqm_ref, akm_ref, v_ref, aq_ref):
    x = x_ref[...]
    q = jnp.dot(x, wq_ref[...], preferred_element_type=jnp.float32) + bq_ref[...]
    k = jnp.dot(x, wk_ref[...], preferred_element_type=jnp.float32) + bk_ref[...]
    v_ref[...] = jnp.dot(x, wv_ref[...], preferred_element_type=jnp.float32) + bv_ref[...]
    aq_ref[...] = (jnp.dot(q, aqm_ref[...], preferred_element_type=jnp.float32)
                   + jnp.dot(k, akm_ref[...], preferred_element_type=jnp.float32))


def _add_body(p_ref, o_ref):
    o_ref[...] = p_ref[0] + p_ref[1]


def _make_edge_kernel(n_nodes, n_edges):
    NW = 32
    K = 80
    e_per = n_edges // NW
    n_chunks = e_per // K
    rchunk = 80
    n_rc = n_nodes // rchunk

    mesh = plsc.VectorSubcoreMesh(core_axis_name="c", subcore_axis_name="s")

    @functools.partial(
        pl.kernel,
        mesh=mesh,
        compiler_params=pltpu.CompilerParams(needs_layout_passes=False,
                                             use_tc_tiling_on_sc=False),
        out_type=jax.ShapeDtypeStruct((2, n_nodes, _H * _C), jnp.float32),
        scratch_types=[
            pltpu.VMEM((K,), jnp.int32),
            pltpu.VMEM((K,), jnp.int32),
            pltpu.VMEM((2 * _H, K), jnp.int32),
            pltpu.VMEM((2 * _H, K), jnp.float32),
            pltpu.VMEM((K, _H * _C), jnp.float32),
            pltpu.VMEM((rchunk, _H * _C), jnp.float32),
            pltpu.VMEM_SHARED((n_nodes, _H * _C), jnp.float32),
            pltpu.SemaphoreType.DMA,
            pltpu.SemaphoreType.DMA,
            pltpu.SemaphoreType.DMA,
        ],
    )
    def edge_kernel(row_hbm, col_hbm, v_hbm, s_hbm, out_hbm,
                    row_v, col_v, si_v, sg_v, vb_v, zb_v, acc,
                    sem0, sem1, sem2):
        cid = lax.axis_index("c")
        sid = lax.axis_index("s")
        wid = cid * 16 + sid

        def _zero(t, carry):
            i = t // 8
            j = t % 8
            zb_v[i, pl.ds(j * 16, 16)] = jnp.zeros((16,), jnp.float32)
            return carry
        lax.fori_loop(0, rchunk * 8, _zero, None)
        for i in range((n_rc + 15) // 16):
            rc = i * 16 + sid

            @pl.when(rc < n_rc)
            def _():
                pltpu.sync_copy(zb_v, acc.at[pl.ds(rc * rchunk, rchunk)])
        plsc.subcore_barrier()

        def _chunk(i, carry):
            base = wid * e_per + i * K
            pltpu.sync_copy(row_hbm.at[pl.ds(base, K)], row_v)
            pltpu.sync_copy(col_hbm.at[pl.ds(base, K)], col_v)
            g3 = pltpu.async_copy(v_hbm.at[col_v], vb_v, sem2)
            for g in range(K // 16):
                sl = pl.ds(g * 16, 16)
                r8 = row_v[sl] * 8
                c8 = col_v[sl] * 8
                for h in range(_H):
                    si_v[h, sl] = r8 + h
                    si_v[_H + h, sl] = c8 + (_H + h)
            gs = [pltpu.async_copy(s_hbm.at[si_v.at[j]], sg_v.at[j], sem0)
                  for j in range(2 * _H)]
            for g_ in gs:
                g_.wait()
            g3.wait()
            for g in range(K // 16):
                e0 = g * 16
                sl = pl.ds(e0, 16)
                ts = []
                for h in range(_H):
                    t = sg_v[h, sl] + sg_v[_H + h, sl]
                    ts.append(jnp.where(t >= 0.0, t, 0.2 * t))
                m = jnp.maximum(jnp.maximum(ts[0], ts[1]),
                                jnp.maximum(ts[2], ts[3]))
                es = [jnp.exp(t - m) for t in ts]
                den = es[0] + es[1] + es[2] + es[3]
                inv = 1.0 / den
                ws = [e_ * inv for e_ in es]

                dnums = lax.GatherDimensionNumbers(
                    offset_dims=(), collapsed_slice_dims=(0,),
                    start_index_map=(0,))

                def _scale(el, carry2):
                    e = e0 + el
                    splat = jnp.full((16, 1), el, jnp.int32)
                    for h in range(_H):
                        a = lax.gather(
                            ws[h], splat, dnums, slice_sizes=(1,),
                            mode=lax.GatherScatterMode.PROMISE_IN_BOUNDS)
                        for j in range(2):
                            c0 = h * _C + j * 16
                            vb_v[e, pl.ds(c0, 16)] = vb_v[e, pl.ds(c0, 16)] * a
                    return carry2
                lax.fori_loop(0, 16, _scale, None)
            pltpu.sync_copy(vb_v, acc.at[row_v], add=True)
            return carry
        lax.fori_loop(0, n_chunks, _chunk, None)

        plsc.subcore_barrier()
        for i in range((n_rc + 15) // 16):
            rc = i * 16 + sid

            @pl.when(rc < n_rc)
            def _():
                sl = pl.ds(rc * rchunk, rchunk)
                pltpu.sync_copy(acc.at[sl], zb_v)
                pltpu.sync_copy(zb_v, out_hbm.at[cid, sl])

    return edge_kernel


def _sample(mu, lv, bmu, blv, key):
    k1, k2 = jax.random.split(key)
    w = mu + jnp.exp(0.5 * lv) * jax.random.normal(k1, mu.shape, dtype=mu.dtype)
    b = bmu + jnp.exp(0.5 * blv) * jax.random.normal(k2, bmu.shape, dtype=bmu.dtype)
    return w, b


def kernel(x, edge_index, wq_mu, wq_lv, bq_mu, bq_lv, wk_mu, wk_lv, bk_mu,
           bk_lv, wv_mu, wv_lv, bv_mu, bv_lv, att):
    n, d_in = x.shape
    out_dim = _H * _C
    n_edges = edge_index.shape[1]

    skey = jax.random.key(42)
    kq, kk, kv = jax.random.split(skey, 3)
    wq, bq = _sample(wq_mu, wq_lv, bq_mu, bq_lv, kq)
    wk, bk = _sample(wk_mu, wk_lv, bk_mu, bk_lv, kk)
    wv, bv = _sample(wv_mu, wv_lv, bv_mu, bv_lv, kv)

    rows = jnp.arange(out_dim)
    heads = rows // _C
    att_q = att[0, :, :_C].reshape(-1)
    att_k = att[0, :, _C:].reshape(-1)
    aq_mat = jnp.zeros((out_dim, 8), jnp.float32).at[rows, heads].set(att_q)
    ak_mat = jnp.zeros((out_dim, 8), jnp.float32).at[rows, heads + _H].set(att_k)

    v, s = pl.pallas_call(
        _proj_body,
        out_shape=(jax.ShapeDtypeStruct((n, out_dim), jnp.float32),
                   jax.ShapeDtypeStruct((n, 8), jnp.float32)),
    )(x, wq.T, bq.reshape(1, -1), wk.T, bk.reshape(1, -1),
      wv.T, bv.reshape(1, -1), aq_mat, ak_mat)

    edge_kernel = _make_edge_kernel(n, n_edges)
    parts = edge_kernel(edge_index[0], edge_index[1], v, s.reshape(-1))

    out = pl.pallas_call(
        _add_body,
        out_shape=jax.ShapeDtypeStruct((n, out_dim), jnp.float32),
    )(parts)
    return out
